# baseline (speedup 1.0000x reference)
import functools
import math

import jax
import jax.numpy as jnp
from jax import lax
from jax.experimental import pallas as pl
from jax.experimental.pallas import tpu as pltpu
from jax.experimental.pallas import tpu_sc as plsc

EMBED_DIM = 32
SQRT_D = math.sqrt(EMBED_DIM)

NC = 2
NS = 16
NW = NC * NS


_NORM_BLOCK = 5000


def _normalize_body(t_ref, o_ref):
    x = t_ref[...]
    jj = lax.broadcasted_iota(jnp.int32, (128, 4), 0) // EMBED_DIM
    kk = lax.broadcasted_iota(jnp.int32, (128, 4), 1)
    sel = (jj == kk).astype(jnp.float32)
    ssq4 = jax.lax.dot_general(
        x * x, sel, (((1,), (0,)), ((), ())),
        precision=jax.lax.Precision.HIGHEST,
        preferred_element_type=jnp.float32,
    )
    scale4 = SQRT_D * lax.rsqrt(jnp.maximum(ssq4, 1e-24))
    scale = jax.lax.dot_general(
        scale4, sel.T, (((1,), (0,)), ((), ())),
        precision=jax.lax.Precision.HIGHEST,
        preferred_element_type=jnp.float32,
    )
    o_ref[...] = x * scale


def _normalize_table(table128):
    n = table128.shape[0]
    grid = n // _NORM_BLOCK
    return pl.pallas_call(
        _normalize_body,
        out_shape=jax.ShapeDtypeStruct(table128.shape, jnp.float32),
        grid=(grid,),
        in_specs=[pl.BlockSpec((_NORM_BLOCK, 128), lambda i: (i, 0))],
        out_specs=pl.BlockSpec((_NORM_BLOCK, 128), lambda i: (i, 0)),
    )(table128)


def _make_gather(batch, hist):
    bw = batch // NW
    btl_n = bw // 128
    c8_n = EMBED_DIM // 8
    bt_n = batch // 128
    mesh = plsc.VectorSubcoreMesh(
        core_axis_name="c", subcore_axis_name="s", num_cores=NC, num_subcores=NS
    )

    @functools.partial(
        pl.kernel,
        out_type=jax.ShapeDtypeStruct((hist, c8_n, bt_n, 8, 128), jnp.float32),
        mesh=mesh,
        scratch_types=[
            pltpu.VMEM((bw,), jnp.int32),
            pltpu.VMEM((bw,), jnp.int32),
            pltpu.VMEM((bw,), jnp.int32),
            pltpu.VMEM((bw, EMBED_DIM), jnp.float32),
            pltpu.VMEM((bw, EMBED_DIM), jnp.float32),
            pltpu.VMEM((bw, EMBED_DIM), jnp.float32),
            pltpu.VMEM((c8_n, btl_n, 8, 133), jnp.float32),
            pltpu.VMEM((c8_n, btl_n, 8, 133), jnp.float32),
            pltpu.SemaphoreType.DMA,
            pltpu.SemaphoreType.DMA,
            pltpu.SemaphoreType.DMA,
            pltpu.SemaphoreType.DMA,
            pltpu.SemaphoreType.DMA,
            pltpu.SemaphoreType.DMA,
            pltpu.SemaphoreType.DMA,
            pltpu.SemaphoreType.DMA,
        ],
        compiler_params=pltpu.CompilerParams(
            use_tc_tiling_on_sc=False, needs_layout_passes=False
        ),
    )
    def gather_k(idsT_hbm, table_hbm, out_hbm, i0, i1, i2, r0, r1, r2, t0, t1,
                 si0, si1, si2, sg0, sg1, sg2, so0, so1):
        wid = lax.axis_index("s") * NC + lax.axis_index("c")
        col0 = wid * bw
        I, R, T = (i0, i1, i2), (r0, r1, r2), (t0, t1)
        SI, SG, SO = (si0, si1, si2), (sg0, sg1, sg2), (so0, so1)
        iot = lax.iota(jnp.int32, 16)

        def idx_start(h, i):
            pltpu.async_copy(idsT_hbm.at[h, pl.ds(col0, bw)], I[i], SI[i])

        def idx_wait(i):
            pltpu.make_async_copy(
                idsT_hbm.at[0, pl.ds(col0, bw)], I[i], SI[i]
            ).wait()

        def gather_start(i):
            pltpu.async_copy(table_hbm.at[I[i]], R[i], SG[i])

        def gather_wait(i):
            pltpu.make_async_copy(table_hbm.at[I[i]], R[i], SG[i]).wait()

        def out_start(h, t):
            for c8 in range(c8_n):
                pltpu.async_copy(
                    T[t].at[c8, :, :, pl.ds(0, 128)],
                    out_hbm.at[h, c8, pl.ds(wid * btl_n, btl_n)],
                    SO[t],
                )

        def out_wait(t):
            for c8 in range(c8_n):
                pltpu.make_async_copy(
                    T[t].at[c8, :, :, pl.ds(0, 128)],
                    out_hbm.at[0, c8, pl.ds(wid * btl_n, btl_n)],
                    SO[t],
                ).wait()

        def transpose_chunk(ri, ti):
            cmv = lax.bitwise_and(iot, 7)
            c8v_lo = lax.shift_right_logical(iot, 3)
            c8v_hi = c8v_lo + 2

            @plsc.parallel_loop(0, bw, step=8, unroll=2)
            def _(q0):
                for j in range(8):
                    q = q0 + j
                    btlv = jnp.full((16,), 0, jnp.int32) + lax.shift_right_logical(q, 7)
                    bmv = jnp.full((16,), 0, jnp.int32) + lax.bitwise_and(q, 127)
                    v_lo = R[ri][q, pl.ds(0, 16)]
                    v_hi = R[ri][q, pl.ds(16, 16)]
                    plsc.store_scatter(T[ti], [c8v_lo, btlv, cmv, bmv], v_lo)
                    plsc.store_scatter(T[ti], [c8v_hi, btlv, cmv, bmv], v_hi)

        def step(h, ri, ti, prefetch=True, start_gather=True, first=False):
            gather_wait(ri)
            if prefetch:
                idx_start(h + 3, ri)
            if start_gather:
                nxt = (ri + 2) % 3
                idx_wait(nxt)
                gather_start(nxt)
            if not first:
                out_wait(ti)
            transpose_chunk(ri, ti)
            out_start(h, ti)

        idx_start(0, 0)
        idx_start(1, 1)
        idx_start(2, 2)
        idx_wait(0)
        gather_start(0)
        idx_wait(1)
        gather_start(1)
        step(0, 0, 0, first=True)
        step(1, 1, 1, first=True)

        def body(g, carry):
            h0 = 6 * g + 2
            for k in range(6):
                step(h0 + k, (2 + k) % 3, k % 2)
            return carry

        lax.fori_loop(0, (hist - 8) // 6, body, 0)

        hb = hist - 6
        for k in range(6):
            h = hb + k
            step(
                h,
                h % 3,
                h % 2,
                prefetch=(h + 3 <= hist - 1),
                start_gather=(h + 2 <= hist - 1),
            )
        out_wait(0)
        out_wait(1)

    return gather_k


def kernel(ids_or_weights, embedding_weight):
    n_rows, d = embedding_weight.shape
    table_n = _normalize_table(
        embedding_weight.reshape(n_rows * d // 128, 128)
    ).reshape(n_rows, d)
    batch, hist = ids_or_weights.shape
    ids_t = ids_or_weights.T
    s = _make_gather(batch, hist)(ids_t, table_n)
    return s.transpose((2, 4, 0, 1, 3)).reshape(batch, hist, EMBED_DIM)


# speedup vs baseline: 1.1171x; 1.1171x over previous
import functools
import math

import jax
import jax.numpy as jnp
from jax import lax
from jax.experimental import pallas as pl
from jax.experimental.pallas import tpu as pltpu
from jax.experimental.pallas import tpu_sc as plsc

EMBED_DIM = 32
SQRT_D = math.sqrt(EMBED_DIM)

NC = 2
NS = 16
NW = NC * NS


_NORM_BLOCK = 5000


def _normalize_body(t_ref, o_ref):
    x = t_ref[...]
    parts = []
    for j in range(128 // EMBED_DIM):
        xs = x[:, j * EMBED_DIM:(j + 1) * EMBED_DIM]
        ssq = jnp.sum(xs * xs, axis=-1, keepdims=True)
        parts.append(xs * (SQRT_D * lax.rsqrt(jnp.maximum(ssq, 1e-24))))
    o_ref[...] = jnp.concatenate(parts, axis=-1)


def _normalize_table(table128):
    n = table128.shape[0]
    grid = n // _NORM_BLOCK
    return pl.pallas_call(
        _normalize_body,
        out_shape=jax.ShapeDtypeStruct(table128.shape, jnp.float32),
        grid=(grid,),
        in_specs=[pl.BlockSpec((_NORM_BLOCK, 128), lambda i: (i, 0))],
        out_specs=pl.BlockSpec((_NORM_BLOCK, 128), lambda i: (i, 0)),
    )(table128)


def _make_gather(batch, hist):
    bw = batch // NW
    btl_n = bw // 128
    c8_n = EMBED_DIM // 8
    bt_n = batch // 128
    mesh = plsc.VectorSubcoreMesh(
        core_axis_name="c", subcore_axis_name="s", num_cores=NC, num_subcores=NS
    )

    @functools.partial(
        pl.kernel,
        out_type=jax.ShapeDtypeStruct((hist, c8_n, bt_n, 8, 128), jnp.float32),
        mesh=mesh,
        scratch_types=[
            pltpu.VMEM((bw,), jnp.int32),
            pltpu.VMEM((bw,), jnp.int32),
            pltpu.VMEM((bw,), jnp.int32),
            pltpu.VMEM((bw, EMBED_DIM), jnp.float32),
            pltpu.VMEM((bw, EMBED_DIM), jnp.float32),
            pltpu.VMEM((bw, EMBED_DIM), jnp.float32),
            pltpu.VMEM((c8_n, btl_n, 8, 133), jnp.float32),
            pltpu.VMEM((c8_n, btl_n, 8, 133), jnp.float32),
            pltpu.SemaphoreType.DMA,
            pltpu.SemaphoreType.DMA,
            pltpu.SemaphoreType.DMA,
            pltpu.SemaphoreType.DMA,
            pltpu.SemaphoreType.DMA,
            pltpu.SemaphoreType.DMA,
            pltpu.SemaphoreType.DMA,
            pltpu.SemaphoreType.DMA,
        ],
        compiler_params=pltpu.CompilerParams(
            use_tc_tiling_on_sc=False, needs_layout_passes=False
        ),
    )
    def gather_k(idsT_hbm, table_hbm, out_hbm, i0, i1, i2, r0, r1, r2, t0, t1,
                 si0, si1, si2, sg0, sg1, sg2, so0, so1):
        wid = lax.axis_index("s") * NC + lax.axis_index("c")
        col0 = wid * bw
        I, R, T = (i0, i1, i2), (r0, r1, r2), (t0, t1)
        SI, SG, SO = (si0, si1, si2), (sg0, sg1, sg2), (so0, so1)
        iot = lax.iota(jnp.int32, 16)

        def idx_start(h, i):
            pltpu.async_copy(idsT_hbm.at[h, pl.ds(col0, bw)], I[i], SI[i])

        def idx_wait(i):
            pltpu.make_async_copy(
                idsT_hbm.at[0, pl.ds(col0, bw)], I[i], SI[i]
            ).wait()

        def gather_start(i):
            pltpu.async_copy(table_hbm.at[I[i]], R[i], SG[i])

        def gather_wait(i):
            pltpu.make_async_copy(table_hbm.at[I[i]], R[i], SG[i]).wait()

        def out_start(h, t):
            for c8 in range(c8_n):
                pltpu.async_copy(
                    T[t].at[c8, :, :, pl.ds(0, 128)],
                    out_hbm.at[h, c8, pl.ds(wid * btl_n, btl_n)],
                    SO[t],
                )

        def out_wait(t):
            for c8 in range(c8_n):
                pltpu.make_async_copy(
                    T[t].at[c8, :, :, pl.ds(0, 128)],
                    out_hbm.at[0, c8, pl.ds(wid * btl_n, btl_n)],
                    SO[t],
                ).wait()

        def transpose_chunk(ri, ti):
            cmv = lax.bitwise_and(iot, 7)
            c8v_lo = lax.shift_right_logical(iot, 3)
            c8v_hi = c8v_lo + 2

            @plsc.parallel_loop(0, bw, step=8, unroll=2)
            def _(q0):
                for j in range(8):
                    q = q0 + j
                    btlv = jnp.full((16,), 0, jnp.int32) + lax.shift_right_logical(q, 7)
                    bmv = jnp.full((16,), 0, jnp.int32) + lax.bitwise_and(q, 127)
                    v_lo = R[ri][q, pl.ds(0, 16)]
                    v_hi = R[ri][q, pl.ds(16, 16)]
                    plsc.store_scatter(T[ti], [c8v_lo, btlv, cmv, bmv], v_lo)
                    plsc.store_scatter(T[ti], [c8v_hi, btlv, cmv, bmv], v_hi)

        def step(h, ri, ti, prefetch=True, start_gather=True, first=False):
            gather_wait(ri)
            if prefetch:
                idx_start(h + 3, ri)
            if start_gather:
                nxt = (ri + 2) % 3
                idx_wait(nxt)
                gather_start(nxt)
            if not first:
                out_wait(ti)
            transpose_chunk(ri, ti)
            out_start(h, ti)

        idx_start(0, 0)
        idx_start(1, 1)
        idx_start(2, 2)
        idx_wait(0)
        gather_start(0)
        idx_wait(1)
        gather_start(1)
        step(0, 0, 0, first=True)
        step(1, 1, 1, first=True)

        def body(g, carry):
            h0 = 6 * g + 2
            for k in range(6):
                step(h0 + k, (2 + k) % 3, k % 2)
            return carry

        lax.fori_loop(0, (hist - 8) // 6, body, 0)

        hb = hist - 6
        for k in range(6):
            h = hb + k
            step(
                h,
                h % 3,
                h % 2,
                prefetch=(h + 3 <= hist - 1),
                start_gather=(h + 2 <= hist - 1),
            )
        out_wait(0)
        out_wait(1)

    return gather_k


def kernel(ids_or_weights, embedding_weight):
    n_rows, d = embedding_weight.shape
    table_n = _normalize_table(
        embedding_weight.reshape(n_rows * d // 128, 128)
    ).reshape(n_rows, d)
    batch, hist = ids_or_weights.shape
    ids_t = ids_or_weights.T
    s = _make_gather(batch, hist)(ids_t, table_n)
    return s.transpose((2, 4, 0, 1, 3)).reshape(batch, hist, EMBED_DIM)
